# baseline (speedup 1.0000x reference)
import functools

import jax
import jax.numpy as jnp
from jax import lax
from jax.experimental import pallas as pl
from jax.experimental.pallas import tpu as pltpu
from jax.experimental.pallas import tpu_sc as plsc

_BLOCK = 128
_NBUF = 4


@functools.lru_cache(maxsize=None)
def _make_sc_lookup(num_rows, num_times, dim):
    info = plsc.get_sparse_core_info()
    nw = info.num_cores * info.num_subcores
    assert num_rows % (nw * _BLOCK) == 0
    n_blocks = num_rows // _BLOCK
    bpw = n_blocks // nw
    assert bpw % _NBUF == 0
    mesh = plsc.VectorSubcoreMesh(core_axis_name="c", subcore_axis_name="s")

    @functools.partial(
        pl.kernel,
        out_type=jax.ShapeDtypeStruct((num_rows, dim), jnp.float32),
        mesh=mesh,
        scratch_types=[
            pltpu.VMEM((bpw, _BLOCK), jnp.int32),
            pltpu.VMEM((bpw, _BLOCK), jnp.int32),
            pltpu.VMEM((_NBUF, _BLOCK, dim), jnp.float32),
            pltpu.VMEM_SHARED((num_times, dim), jnp.float32),
            pltpu.SemaphoreType.DMA((_NBUF,)),
            pltpu.SemaphoreType.DMA((_NBUF,)),
            pltpu.SemaphoreType.DMA((_NBUF,)),
        ],
        compiler_params=pltpu.CompilerParams(use_tc_tiling_on_sc=False,
                                             needs_layout_passes=False),
    )
    def sc_lookup(idx_hbm, hour_hbm, item_hbm, time_hbm, out_hbm,
                  idx_v, hour_v, rows_v, time_s, sem_i, sem_t, sem_w):
        wid = lax.axis_index("s") * info.num_cores + lax.axis_index("c")
        base_blk = wid * bpw
        pltpu.sync_copy(idx_hbm.at[pl.ds(base_blk, bpw)], idx_v)
        pltpu.sync_copy(hour_hbm.at[pl.ds(base_blk, bpw)], hour_v)
        pltpu.sync_copy(time_hbm, time_s)

        def i_copy(j, b):
            return pltpu.make_async_copy(
                item_hbm.at[idx_v.at[j]], rows_v.at[b], sem_i.at[b])

        def t_copy(j, b):
            return pltpu.make_async_copy(
                time_s.at[hour_v.at[j]], rows_v.at[b], sem_t.at[b])

        def w_copy(j, b):
            return pltpu.make_async_copy(
                rows_v.at[b],
                out_hbm.at[pl.ds((base_blk + j) * _BLOCK, _BLOCK)],
                sem_w.at[b])

        def group(g, carry):
            for b in range(_NBUF):
                j = g * _NBUF + b

                @pl.when(j < bpw)
                def _i():
                    @pl.when(j >= _NBUF)
                    def _():
                        w_copy(j - _NBUF, b).wait()
                    i_copy(j, b).start()

                jt, bt = j - 1, (b - 1) % _NBUF

                @pl.when(jnp.logical_and(jt >= 0, jt < bpw))
                def _t():
                    i_copy(jt, bt).wait()
                    pltpu.async_copy(time_s.at[hour_v.at[jt]],
                                     rows_v.at[bt], sem_t.at[bt], add=True)

                jw, bw = j - 2, (b - 2) % _NBUF

                @pl.when(jnp.logical_and(jw >= 0, jw < bpw))
                def _w():
                    t_copy(jw, bw).wait()
                    w_copy(jw, bw).start()

            return carry

        lax.fori_loop(0, bpw // _NBUF + 1, group, 0)
        for b in range(_NBUF):
            w_copy(bpw - _NBUF + b, b).wait()

    return sc_lookup


def kernel(item_ids, hour_of_day, item_table, time_table):
    batch, hist = item_ids.shape
    num_rows = batch * hist
    dim = item_table.shape[1]
    idx2 = item_ids.reshape(num_rows // _BLOCK, _BLOCK).astype(jnp.int32)
    hour2 = hour_of_day.reshape(num_rows // _BLOCK, _BLOCK).astype(jnp.int32)
    fn = _make_sc_lookup(num_rows, time_table.shape[0], dim)
    out = fn(idx2, hour2, item_table, time_table)
    return out.reshape(batch, hist, dim)
